# baseline (speedup 1.0000x reference)
import jax
import jax.numpy as jnp
from jax.experimental import pallas as pl
from jax.experimental.pallas import tpu as pltpu

_B = 32
_HALF = 16
_BC = 8
_NBLK = _B // _BC
_NHALF = _NBLK // 2
_P = 8732
_C = 21
_EPS = 1e-7


def _loss_body(c_ref, q_ref, num_ref, cnt_ref, kl_scr, left_scr):
    g = pl.program_id(0)

    @pl.when(g == 0)
    def _init():
        num_ref[...] = jnp.zeros_like(num_ref)
        cnt_ref[...] = jnp.zeros_like(cnt_ref)

    bg = c_ref[0]
    t = bg + _EPS
    kl_row = t * (jnp.log(t) - jnp.log(q_ref[0] + _EPS))
    cmax = c_ref[1]
    t = cmax + _EPS
    kl_row += t * (jnp.log(t) - jnp.log(q_ref[1] + _EPS))
    for cls in range(2, _C):
        v = c_ref[cls]
        cmax = jnp.maximum(cmax, v)
        t = v + _EPS
        kl_row += t * (jnp.log(t) - jnp.log(q_ref[cls] + _EPS))

    l1 = cmax > bg

    @pl.when(g < _NHALF)
    def _store():
        kl_scr[pl.ds(g * _BC, _BC), :] = kl_row
        left_scr[pl.ds(g * _BC, _BC), :] = jnp.where(l1, 1.0, 0.0)

    @pl.when(g >= _NHALF)
    def _reduce():
        off = (g - _NHALF) * _BC
        k0 = kl_scr[pl.ds(off, _BC), :]
        l0 = left_scr[pl.ds(off, _BC), :] > 0.5
        m0 = jnp.logical_and(l0, jnp.logical_not(l1))
        m1 = jnp.logical_and(l1, jnp.logical_not(l0))
        part = jnp.sum(jnp.where(m0, k0, 0.0)) + jnp.sum(
            jnp.where(m1, kl_row, 0.0))
        num_ref[...] += jnp.full((1, 1), part)
        cnt_ref[...] += jnp.full((1, 1), jnp.sum(jnp.where(m0, 1.0, 0.0))
                                 + jnp.sum(jnp.where(m1, 1.0, 0.0)))


def kernel(args, lam, conf, loc, conf_mix, loc_mix):
    del args, lam, loc, loc_mix
    conf_t = jnp.transpose(conf, (2, 0, 1))
    mix_t = jnp.transpose(conf_mix, (2, 0, 1))
    in_spec = pl.BlockSpec((_C, _BC, _P), lambda g: (0, g, 0))
    out_spec = pl.BlockSpec((1, 1), lambda g: (0, 0))
    num, cnt = pl.pallas_call(
        _loss_body,
        grid=(_NBLK,),
        in_specs=[in_spec, in_spec],
        out_specs=[out_spec, out_spec],
        out_shape=[
            jax.ShapeDtypeStruct((1, 1), jnp.float32),
            jax.ShapeDtypeStruct((1, 1), jnp.float32),
        ],
        scratch_shapes=[
            pltpu.VMEM((_HALF, _P), jnp.float32),
            pltpu.VMEM((_HALF, _P), jnp.float32),
        ],
    )(conf_t, mix_t)
    num = num[0, 0]
    cnt = cnt[0, 0]
    loss = jnp.where(cnt > 0, num / jnp.maximum(cnt, 1.0), jnp.float32(0.0))
    return (jnp.zeros((1,), dtype=jnp.float32), loss)


# speedup vs baseline: 1.0793x; 1.0793x over previous
import jax
import jax.numpy as jnp
from jax.experimental import pallas as pl
from jax.experimental.pallas import tpu as pltpu

_B = 32
_HALF = 16
_BC = 8
_NBLK = _B // _BC
_P = 8732
_C = 21
_EPS = 1e-7


def _loss_body(c_ref, q_ref, num_ref, cnt_ref, kl_scr, left_scr):
    g = pl.program_id(0)

    bg = c_ref[0]
    t = bg + _EPS
    kl_row = t * (jnp.log(t) - jnp.log(q_ref[0] + _EPS))
    cmax = c_ref[1]
    t = cmax + _EPS
    kl_row += t * (jnp.log(t) - jnp.log(q_ref[1] + _EPS))
    for cls in range(2, _C):
        v = c_ref[cls]
        cmax = jnp.maximum(cmax, v)
        t = v + _EPS
        kl_row += t * (jnp.log(t) - jnp.log(q_ref[cls] + _EPS))

    kl_scr[pl.ds(g * _BC, _BC), :] = kl_row
    left_scr[pl.ds(g * _BC, _BC), :] = jnp.where(cmax > bg, 1.0, 0.0)

    @pl.when(g == _NBLK - 1)
    def _reduce():
        left = left_scr[...] > 0.5
        right = jnp.concatenate([left[_HALF:], left[:_HALF]], axis=0)
        m = jnp.logical_and(left, jnp.logical_not(right))
        num_ref[...] = jnp.full((1, 1), jnp.sum(jnp.where(m, kl_scr[...], 0.0)))
        cnt_ref[...] = jnp.full((1, 1), jnp.sum(jnp.where(m, 1.0, 0.0)))


def kernel(args, lam, conf, loc, conf_mix, loc_mix):
    del args, lam, loc, loc_mix
    conf_t = jnp.transpose(conf, (2, 0, 1))
    mix_t = jnp.transpose(conf_mix, (2, 0, 1))
    in_spec = pl.BlockSpec((_C, _BC, _P), lambda g: (0, g, 0))
    out_spec = pl.BlockSpec((1, 1), lambda g: (0, 0))
    num, cnt = pl.pallas_call(
        _loss_body,
        grid=(_NBLK,),
        in_specs=[in_spec, in_spec],
        out_specs=[out_spec, out_spec],
        out_shape=[
            jax.ShapeDtypeStruct((1, 1), jnp.float32),
            jax.ShapeDtypeStruct((1, 1), jnp.float32),
        ],
        scratch_shapes=[
            pltpu.VMEM((_B, _P), jnp.float32),
            pltpu.VMEM((_B, _P), jnp.float32),
        ],
    )(conf_t, mix_t)
    num = num[0, 0]
    cnt = cnt[0, 0]
    loss = jnp.where(cnt > 0, num / jnp.maximum(cnt, 1.0), jnp.float32(0.0))
    return (jnp.zeros((1,), dtype=jnp.float32), loss)
